# Initial kernel scaffold; baseline (speedup 1.0000x reference)
import jax
import jax.numpy as jnp
from jax.experimental import pallas as pl


def kernel(xyz, new_xyz, features):
    raise NotImplementedError("write your pallas kernel here")


# speedup vs baseline: 12.5970x; 12.5970x over previous
import functools

import jax
import jax.numpy as jnp
from jax import lax
from jax.experimental import pallas as pl
from jax.experimental.pallas import tpu as pltpu
from jax.experimental.pallas import tpu_sc as plsc

RADIUS = 0.25
NSAMPLE = 32
B, N, P, C = 4, 8192, 1024, 128
PS = P * NSAMPLE

NC, NS = 2, 16
NW = NC * NS
ROWS_PER_W = (B * P) // NW
SEL_CHUNK = 8
GQ = 8192


def _mask_body(new_ref, xyzt_ref, mask_ref):
    nxyz = new_ref[0]
    x = xyzt_ref[0]
    n2 = (nxyz[:, 0:1] * nxyz[:, 0:1] + nxyz[:, 1:2] * nxyz[:, 1:2]
          + nxyz[:, 2:3] * nxyz[:, 2:3])
    x2 = (x[0:1] * x[0:1] + x[1:2] * x[1:2] + x[2:3] * x[2:3])
    inner = jnp.dot(nxyz, x, preferred_element_type=jnp.float32)
    dist2 = n2 + x2 - 2.0 * inner
    within = dist2 <= (RADIUS * RADIUS)
    mask_ref[0] = within.astype(jnp.int32)


def _compute_mask(new_xyz, xyz_t):
    NB = 2048
    grid = (B, N // NB)
    return pl.pallas_call(
        _mask_body,
        grid=grid,
        in_specs=[
            pl.BlockSpec((1, P, 3), lambda b, n: (b, 0, 0)),
            pl.BlockSpec((1, 3, NB), lambda b, n: (b, 0, n)),
        ],
        out_specs=pl.BlockSpec((1, P, NB), lambda b, n: (b, 0, n)),
        out_shape=jax.ShapeDtypeStruct((B, P, N), jnp.int32),
    )(new_xyz, xyz_t)


def _select_body(mask_hbm, bq_hbm, im_hbm, mbuf, bqb, imb, idxbuf):
    cid = lax.axis_index("c")
    sid = lax.axis_index("s")
    wid = sid * NC + cid
    row0 = wid * ROWS_PER_W
    ii = lax.iota(jnp.int32, 16)

    def chunk_loop(ci, _):
        r0 = row0 + ci * SEL_CHUNK
        pltpu.sync_copy(mask_hbm.at[pl.ds(r0, SEL_CHUNK), :], mbuf)

        def row_loop(rj, _):
            def do_block(k0, cnt):
                for u in range(8):
                    v = mbuf[rj, pl.ds((k0 * 8 + u) * 16, 16)]
                    m = v > 0
                    pos = (k0 * 8 + u) * 16 + ii
                    off = jnp.minimum(cnt, 144)
                    plsc.store_compressed(idxbuf.at[pl.ds(off, 16)], pos,
                                          mask=m)
                    pc = plsc.all_reduce_population_count(m)
                    cnt = cnt + pc[0]
                return cnt

            def blk_loop(k0, cnt):
                return lax.cond(cnt < NSAMPLE,
                                lambda: do_block(k0, cnt),
                                lambda: cnt)

            cnt = lax.fori_loop(0, 64, blk_loop, jnp.int32(0))
            cnt_c = jnp.minimum(cnt, NSAMPLE)
            head = idxbuf[pl.ds(0, 16)]
            first = jnp.where(cnt > 0, head[0], 0)
            eff = jnp.maximum(cnt_c, 1)
            for s2 in range(2):
                sl = ii + (16 * s2)
                cur = idxbuf[pl.ds(16 * s2, 16)]
                fsp = jnp.broadcast_to(first, (16,))
                bqv = jnp.where(sl < cnt_c, cur, fsp)
                imv = jnp.where(sl < eff, bqv, N)
                rloc = ci * SEL_CHUNK + rj
                bqb[rloc, pl.ds(16 * s2, 16)] = bqv
                imb[rloc, pl.ds(16 * s2, 16)] = imv
            return 0

        lax.fori_loop(0, SEL_CHUNK, row_loop, 0)
        return 0

    lax.fori_loop(0, ROWS_PER_W // SEL_CHUNK, chunk_loop, 0)
    pltpu.sync_copy(bqb, bq_hbm.at[pl.ds(row0, ROWS_PER_W), :])
    pltpu.sync_copy(imb, im_hbm.at[pl.ds(row0, ROWS_PER_W), :])


def _select(mask2d):
    mesh = plsc.VectorSubcoreMesh(core_axis_name="c", subcore_axis_name="s", num_cores=NC, num_subcores=NS)
    return pl.kernel(
        _select_body,
        compiler_params=pltpu.CompilerParams(needs_layout_passes=False),
        out_type=(
            jax.ShapeDtypeStruct((B * P, NSAMPLE), jnp.int32),
            jax.ShapeDtypeStruct((B * P, NSAMPLE), jnp.int32),
        ),
        mesh=mesh,
        scratch_types=[
            pltpu.VMEM((SEL_CHUNK, N // 16 * 16), jnp.int32),
            pltpu.VMEM((ROWS_PER_W, NSAMPLE), jnp.int32),
            pltpu.VMEM((ROWS_PER_W, NSAMPLE), jnp.int32),
            pltpu.VMEM((176,), jnp.int32),
        ],
    )(mask2d)


def _gather_body(feat_hbm, xyzt_hbm, rep_hbm, idx_hbm,
                 nf_hbm, gx_hbm,
                 idxv, rowbuf, repbuf, outbuf, gxbuf):
    cid = lax.axis_index("c")
    sid = lax.axis_index("s")
    wid = sid * NC + cid
    b = wid // 8
    s8 = wid % 8
    nrows = jnp.where(s8 < 3, 17, 16)
    rstart = 16 * s8 + jnp.minimum(s8, 3)

    pltpu.sync_copy(idx_hbm.at[b], idxv)

    def row_loop(ri, _):
        r = rstart + ri

        @pl.when(r < 3)
        def _xyz_row():
            pltpu.sync_copy(xyzt_hbm.at[b, r], rowbuf.at[pl.ds(0, N)])
            rowbuf[pl.ds(N, 16)] = jnp.full((16,), 1000000.0, jnp.float32)

            def q_loop(q, _):
                pltpu.sync_copy(rep_hbm.at[b, r, pl.ds(q * GQ, GQ)], repbuf)

                def j_loop(j0, _):
                    for u in range(8):
                        j = j0 * 8 + u
                        iv = idxv[pl.ds(q * GQ + j * 16, 16)]
                        g = plsc.load_gather(rowbuf, [iv])
                        rv = repbuf[pl.ds(j * 16, 16)]
                        gxv = g - rv
                        gxbuf[pl.ds(j * 16, 16)] = gxv
                        nfv = jnp.where(gxv > 100000.0, 0.0, gxv) * 4.0
                        outbuf[pl.ds(j * 16, 16)] = nfv
                    return 0

                lax.fori_loop(0, GQ // 128, j_loop, 0)
                pltpu.sync_copy(outbuf, nf_hbm.at[b, r, pl.ds(q * GQ, GQ)])
                pltpu.sync_copy(gxbuf, gx_hbm.at[b, r, pl.ds(q * GQ, GQ)])
                return 0

            lax.fori_loop(0, PS // GQ, q_loop, 0)

        @pl.when(r >= 3)
        def _feat_row():
            pltpu.sync_copy(feat_hbm.at[b, r - 3], rowbuf.at[pl.ds(0, N)])
            rowbuf[pl.ds(N, 16)] = jnp.zeros((16,), jnp.float32)

            def q_loop(q, _):
                def j_loop(j0, _):
                    for u in range(8):
                        j = j0 * 8 + u
                        iv = idxv[pl.ds(q * GQ + j * 16, 16)]
                        g = plsc.load_gather(rowbuf, [iv])
                        outbuf[pl.ds(j * 16, 16)] = g
                    return 0

                lax.fori_loop(0, GQ // 128, j_loop, 0)
                pltpu.sync_copy(outbuf, nf_hbm.at[b, r, pl.ds(q * GQ, GQ)])
                return 0

            lax.fori_loop(0, PS // GQ, q_loop, 0)

        return 0

    lax.fori_loop(0, nrows, row_loop, 0)


def _gather(features, xyz_t, rep, idxm):
    mesh = plsc.VectorSubcoreMesh(core_axis_name="c", subcore_axis_name="s", num_cores=NC, num_subcores=NS)
    return pl.kernel(
        _gather_body,
        compiler_params=pltpu.CompilerParams(needs_layout_passes=False),
        out_type=(
            jax.ShapeDtypeStruct((B, 3 + C, PS), jnp.float32),
            jax.ShapeDtypeStruct((B, 3, PS), jnp.float32),
        ),
        mesh=mesh,
        scratch_types=[
            pltpu.VMEM((PS,), jnp.int32),
            pltpu.VMEM((N + 16,), jnp.float32),
            pltpu.VMEM((GQ,), jnp.float32),
            pltpu.VMEM((GQ,), jnp.float32),
            pltpu.VMEM((GQ,), jnp.float32),
        ],
    )(features, xyz_t, rep, idxm)


@jax.jit
def kernel(xyz, new_xyz, features):
    xyz_t = jnp.transpose(xyz, (0, 2, 1))
    mask = _compute_mask(new_xyz, xyz_t)
    bq, idxm = _select(mask.reshape(B * P, N))
    rep = jnp.broadcast_to(
        jnp.transpose(new_xyz, (0, 2, 1))[..., None],
        (B, 3, P, NSAMPLE)).reshape(B, 3, PS)
    nf3, gx3 = _gather(features, xyz_t, rep, idxm.reshape(B, PS))
    new_features = nf3.reshape(B, 3 + C, P, NSAMPLE)
    grouped_xyz = gx3.reshape(B, 3, P, NSAMPLE)
    bq_idx = bq.reshape(B, P, NSAMPLE).astype(jnp.int64)
    return (new_features, grouped_xyz, bq_idx)


# speedup vs baseline: 12.6127x; 1.0012x over previous
import functools

import jax
import jax.numpy as jnp
from jax import lax
from jax.experimental import pallas as pl
from jax.experimental.pallas import tpu as pltpu
from jax.experimental.pallas import tpu_sc as plsc

RADIUS = 0.25
NSAMPLE = 32
B, N, P, C = 4, 8192, 1024, 128
PS = P * NSAMPLE

NC, NS = 2, 16
NW = NC * NS
ROWS_PER_W = (B * P) // NW
SEL_CHUNK = 8
GQ = 8192


def _mask_body(new_ref, xyzt_ref, mask_ref):
    nxyz = new_ref[0]
    x = xyzt_ref[0]
    n2 = (nxyz[:, 0:1] * nxyz[:, 0:1] + nxyz[:, 1:2] * nxyz[:, 1:2]
          + nxyz[:, 2:3] * nxyz[:, 2:3])
    x2 = (x[0:1] * x[0:1] + x[1:2] * x[1:2] + x[2:3] * x[2:3])
    inner = jnp.dot(nxyz, x, preferred_element_type=jnp.float32)
    dist2 = n2 + x2 - 2.0 * inner
    within = dist2 <= (RADIUS * RADIUS)
    mask_ref[...] = within.astype(jnp.int32)


def _compute_mask(new_xyz, xyz_t):
    NB = 2048
    grid = (B, N // NB)
    return pl.pallas_call(
        _mask_body,
        grid=grid,
        in_specs=[
            pl.BlockSpec((1, P, 3), lambda b, n: (b, 0, 0)),
            pl.BlockSpec((1, 3, NB), lambda b, n: (b, 0, n)),
        ],
        out_specs=pl.BlockSpec((P, NB), lambda b, n: (b, n)),
        out_shape=jax.ShapeDtypeStruct((B * P, N), jnp.int32),
    )(new_xyz, xyz_t)


def _select_body(mask_hbm, bq_hbm, im_hbm, mbuf, bqb, imb, idxbuf):
    cid = lax.axis_index("c")
    sid = lax.axis_index("s")
    wid = sid * NC + cid
    row0 = wid * ROWS_PER_W
    ii = lax.iota(jnp.int32, 16)

    def chunk_loop(ci, _):
        r0 = row0 + ci * SEL_CHUNK
        pltpu.sync_copy(mask_hbm.at[pl.ds(r0, SEL_CHUNK), :], mbuf)

        def row_loop(rj, _):
            def do_block(k0, cnt):
                for u in range(8):
                    v = mbuf[rj, pl.ds((k0 * 8 + u) * 16, 16)]
                    m = v > 0
                    pos = (k0 * 8 + u) * 16 + ii
                    off = jnp.minimum(cnt, 144)
                    plsc.store_compressed(idxbuf.at[pl.ds(off, 16)], pos,
                                          mask=m)
                    pc = plsc.all_reduce_population_count(m)
                    cnt = cnt + pc[0]
                return cnt

            def blk_loop(k0, cnt):
                return lax.cond(cnt < NSAMPLE,
                                lambda: do_block(k0, cnt),
                                lambda: cnt)

            cnt = lax.fori_loop(0, 64, blk_loop, jnp.int32(0))
            cnt_c = jnp.minimum(cnt, NSAMPLE)
            head = idxbuf[pl.ds(0, 16)]
            first = jnp.where(cnt > 0, head[0], 0)
            eff = jnp.maximum(cnt_c, 1)
            for s2 in range(2):
                sl = ii + (16 * s2)
                cur = idxbuf[pl.ds(16 * s2, 16)]
                fsp = jnp.broadcast_to(first, (16,))
                bqv = jnp.where(sl < cnt_c, cur, fsp)
                imv = jnp.where(sl < eff, bqv, N)
                rloc = ci * SEL_CHUNK + rj
                bqb[rloc, pl.ds(16 * s2, 16)] = bqv
                imb[pl.ds(rloc * NSAMPLE + 16 * s2, 16)] = imv
            return 0

        lax.fori_loop(0, SEL_CHUNK, row_loop, 0)
        return 0

    lax.fori_loop(0, ROWS_PER_W // SEL_CHUNK, chunk_loop, 0)
    b_w = row0 // P
    p0 = row0 % P
    pltpu.sync_copy(bqb, bq_hbm.at[b_w, pl.ds(p0, ROWS_PER_W), :])
    pltpu.sync_copy(imb, im_hbm.at[b_w, pl.ds(p0 * NSAMPLE,
                                              ROWS_PER_W * NSAMPLE)])


def _select(mask2d):
    mesh = plsc.VectorSubcoreMesh(core_axis_name="c", subcore_axis_name="s", num_cores=NC, num_subcores=NS)
    return pl.kernel(
        _select_body,
        compiler_params=pltpu.CompilerParams(needs_layout_passes=False),
        out_type=(
            jax.ShapeDtypeStruct((B, P, NSAMPLE), jnp.int32),
            jax.ShapeDtypeStruct((B, PS), jnp.int32),
        ),
        mesh=mesh,
        scratch_types=[
            pltpu.VMEM((SEL_CHUNK, N), jnp.int32),
            pltpu.VMEM((ROWS_PER_W, NSAMPLE), jnp.int32),
            pltpu.VMEM((ROWS_PER_W * NSAMPLE,), jnp.int32),
            pltpu.VMEM((176,), jnp.int32),
        ],
    )(mask2d)


def _gather_body(feat_hbm, xyzt_hbm, rep_hbm, idx_hbm,
                 nf_hbm, gx_hbm,
                 idxv, rowbuf, repbuf, outbuf, gxbuf):
    cid = lax.axis_index("c")
    sid = lax.axis_index("s")
    wid = sid * NC + cid
    b = wid // 8
    s8 = wid % 8
    nrows = jnp.where(s8 < 3, 17, 16)
    rstart = 16 * s8 + jnp.minimum(s8, 3)

    pltpu.sync_copy(idx_hbm.at[b], idxv)

    def row_loop(ri, _):
        r = rstart + ri

        @pl.when(r < 3)
        def _xyz_row():
            pltpu.sync_copy(xyzt_hbm.at[b, r], rowbuf.at[pl.ds(0, N)])
            rowbuf[pl.ds(N, 16)] = jnp.full((16,), 1000000.0, jnp.float32)

            def q_loop(q, _):
                pltpu.sync_copy(rep_hbm.at[b, r, pl.ds(q * GQ, GQ)], repbuf)

                def j_loop(j0, _):
                    for u in range(8):
                        j = j0 * 8 + u
                        iv = idxv[pl.ds(q * GQ + j * 16, 16)]
                        g = plsc.load_gather(rowbuf, [iv])
                        rv = repbuf[pl.ds(j * 16, 16)]
                        gxv = g - rv
                        gxbuf[pl.ds(j * 16, 16)] = gxv
                        nfv = jnp.where(gxv > 100000.0, 0.0, gxv) * 4.0
                        outbuf[pl.ds(j * 16, 16)] = nfv
                    return 0

                lax.fori_loop(0, GQ // 128, j_loop, 0)
                pltpu.sync_copy(outbuf, nf_hbm.at[b, r, pl.ds(q * GQ, GQ)])
                pltpu.sync_copy(gxbuf, gx_hbm.at[b, r, pl.ds(q * GQ, GQ)])
                return 0

            lax.fori_loop(0, PS // GQ, q_loop, 0)

        @pl.when(r >= 3)
        def _feat_row():
            pltpu.sync_copy(feat_hbm.at[b, r - 3], rowbuf.at[pl.ds(0, N)])
            rowbuf[pl.ds(N, 16)] = jnp.zeros((16,), jnp.float32)

            def q_loop(q, _):
                def j_loop(j0, _):
                    for u in range(8):
                        j = j0 * 8 + u
                        iv = idxv[pl.ds(q * GQ + j * 16, 16)]
                        g = plsc.load_gather(rowbuf, [iv])
                        outbuf[pl.ds(j * 16, 16)] = g
                    return 0

                lax.fori_loop(0, GQ // 128, j_loop, 0)
                pltpu.sync_copy(outbuf, nf_hbm.at[b, r, pl.ds(q * GQ, GQ)])
                return 0

            lax.fori_loop(0, PS // GQ, q_loop, 0)

        return 0

    lax.fori_loop(0, nrows, row_loop, 0)


def _gather(features, xyz_t, rep, idxm):
    mesh = plsc.VectorSubcoreMesh(core_axis_name="c", subcore_axis_name="s", num_cores=NC, num_subcores=NS)
    return pl.kernel(
        _gather_body,
        compiler_params=pltpu.CompilerParams(needs_layout_passes=False),
        out_type=(
            jax.ShapeDtypeStruct((B, 3 + C, PS), jnp.float32),
            jax.ShapeDtypeStruct((B, 3, PS), jnp.float32),
        ),
        mesh=mesh,
        scratch_types=[
            pltpu.VMEM((PS,), jnp.int32),
            pltpu.VMEM((N + 16,), jnp.float32),
            pltpu.VMEM((GQ,), jnp.float32),
            pltpu.VMEM((GQ,), jnp.float32),
            pltpu.VMEM((GQ,), jnp.float32),
        ],
    )(features, xyz_t, rep, idxm)


@jax.jit
def kernel(xyz, new_xyz, features):
    xyz_t = jnp.transpose(xyz, (0, 2, 1))
    mask = _compute_mask(new_xyz, xyz_t)
    bq, idxm = _select(mask)
    rep = jnp.broadcast_to(
        jnp.transpose(new_xyz, (0, 2, 1))[..., None],
        (B, 3, P, NSAMPLE)).reshape(B, 3, PS)
    nf3, gx3 = _gather(features, xyz_t, rep, idxm)
    new_features = nf3.reshape(B, 3 + C, P, NSAMPLE)
    grouped_xyz = gx3.reshape(B, 3, P, NSAMPLE)
    bq_idx = bq.astype(jnp.int64)
    return (new_features, grouped_xyz, bq_idx)


# speedup vs baseline: 14.6549x; 1.1619x over previous
import functools

import jax
import jax.numpy as jnp
from jax import lax
from jax.experimental import pallas as pl
from jax.experimental.pallas import tpu as pltpu
from jax.experimental.pallas import tpu_sc as plsc

RADIUS = 0.25
NSAMPLE = 32
B, N, P, C = 4, 8192, 1024, 128
PS = P * NSAMPLE

NC, NS = 2, 16
NW = NC * NS
ROWS_PER_W = (B * P) // NW
SEL_CHUNK = 8
GQ = 8192


def _mask_body(new_ref, xyzt_ref, mask_ref):
    nxyz = new_ref[0]
    x = xyzt_ref[0]
    n2 = (nxyz[:, 0:1] * nxyz[:, 0:1] + nxyz[:, 1:2] * nxyz[:, 1:2]
          + nxyz[:, 2:3] * nxyz[:, 2:3])
    x2 = (x[0:1] * x[0:1] + x[1:2] * x[1:2] + x[2:3] * x[2:3])
    inner = jnp.dot(nxyz, x, preferred_element_type=jnp.float32)
    dist2 = n2 + x2 - 2.0 * inner
    within = dist2 <= (RADIUS * RADIUS)
    mask_ref[...] = within.astype(jnp.int32)


def _compute_mask(new_xyz, xyz_t):
    NB = 2048
    grid = (B, N // NB)
    return pl.pallas_call(
        _mask_body,
        grid=grid,
        in_specs=[
            pl.BlockSpec((1, P, 3), lambda b, n: (b, 0, 0)),
            pl.BlockSpec((1, 3, NB), lambda b, n: (b, 0, n)),
        ],
        out_specs=pl.BlockSpec((P, NB), lambda b, n: (b, n)),
        out_shape=jax.ShapeDtypeStruct((B * P, N), jnp.int32),
    )(new_xyz, xyz_t)


def _select_body(mask_hbm, bq_hbm, im_hbm, mbuf, bqb, imb, idxbuf):
    cid = lax.axis_index("c")
    sid = lax.axis_index("s")
    wid = sid * NC + cid
    row0 = wid * ROWS_PER_W
    ii = lax.iota(jnp.int32, 16)

    def chunk_loop(ci, _):
        r0 = row0 + ci * SEL_CHUNK
        pltpu.sync_copy(mask_hbm.at[pl.ds(r0, SEL_CHUNK), :], mbuf)

        def row_loop(rj, _):
            def do_block(k0, cnt):
                vs = [mbuf[rj, pl.ds((k0 * 8 + u) * 16, 16)]
                      for u in range(8)]
                t01 = vs[0] | vs[1]
                t23 = vs[2] | vs[3]
                t45 = vs[4] | vs[5]
                t67 = vs[6] | vs[7]
                t = (t01 | t23) | (t45 | t67)

                def extract():
                    c = cnt
                    for u in range(8):
                        m = vs[u] > 0
                        pos = (k0 * 8 + u) * 16 + ii

                        def one_vreg(c=c, m=m, pos=pos):
                            off = jnp.minimum(c, 144)
                            plsc.store_compressed(
                                idxbuf.at[pl.ds(off, 16)], pos, mask=m)
                            pc = plsc.all_reduce_population_count(m)
                            return c + pc[0]

                        c = lax.cond(jnp.any(m), one_vreg, lambda c=c: c)
                    return c

                return lax.cond(jnp.any(t > 0), extract, lambda: cnt)

            def blk_loop(k0, cnt):
                return lax.cond(cnt < NSAMPLE,
                                lambda: do_block(k0, cnt),
                                lambda: cnt)

            cnt = lax.fori_loop(0, 64, blk_loop, jnp.int32(0))
            cnt_c = jnp.minimum(cnt, NSAMPLE)
            head = idxbuf[pl.ds(0, 16)]
            first = jnp.where(cnt > 0, head[0], 0)
            eff = jnp.maximum(cnt_c, 1)
            for s2 in range(2):
                sl = ii + (16 * s2)
                cur = idxbuf[pl.ds(16 * s2, 16)]
                fsp = jnp.broadcast_to(first, (16,))
                bqv = jnp.where(sl < cnt_c, cur, fsp)
                imv = jnp.where(sl < eff, bqv, N)
                rloc = ci * SEL_CHUNK + rj
                bqb[rloc, pl.ds(16 * s2, 16)] = bqv
                imb[pl.ds(rloc * NSAMPLE + 16 * s2, 16)] = imv
            return 0

        lax.fori_loop(0, SEL_CHUNK, row_loop, 0)
        return 0

    lax.fori_loop(0, ROWS_PER_W // SEL_CHUNK, chunk_loop, 0)
    b_w = row0 // P
    p0 = row0 % P
    pltpu.sync_copy(bqb, bq_hbm.at[b_w, pl.ds(p0, ROWS_PER_W), :])
    pltpu.sync_copy(imb, im_hbm.at[b_w, pl.ds(p0 * NSAMPLE,
                                              ROWS_PER_W * NSAMPLE)])


def _select(mask2d):
    mesh = plsc.VectorSubcoreMesh(core_axis_name="c", subcore_axis_name="s", num_cores=NC, num_subcores=NS)
    return pl.kernel(
        _select_body,
        compiler_params=pltpu.CompilerParams(needs_layout_passes=False),
        out_type=(
            jax.ShapeDtypeStruct((B, P, NSAMPLE), jnp.int32),
            jax.ShapeDtypeStruct((B, PS), jnp.int32),
        ),
        mesh=mesh,
        scratch_types=[
            pltpu.VMEM((SEL_CHUNK, N), jnp.int32),
            pltpu.VMEM((ROWS_PER_W, NSAMPLE), jnp.int32),
            pltpu.VMEM((ROWS_PER_W * NSAMPLE,), jnp.int32),
            pltpu.VMEM((176,), jnp.int32),
        ],
    )(mask2d)


def _gather_body(feat_hbm, xyzt_hbm, rep_hbm, idx_hbm,
                 nf_hbm, gx_hbm,
                 idxv, rowbuf, repbuf, outbuf, gxbuf):
    cid = lax.axis_index("c")
    sid = lax.axis_index("s")
    wid = sid * NC + cid
    b = wid // 8
    s8 = wid % 8
    nrows = jnp.where(s8 < 3, 17, 16)
    rstart = 16 * s8 + jnp.minimum(s8, 3)

    pltpu.sync_copy(idx_hbm.at[b], idxv)

    def row_loop(ri, _):
        r = rstart + ri

        @pl.when(r < 3)
        def _xyz_row():
            pltpu.sync_copy(xyzt_hbm.at[b, r], rowbuf.at[pl.ds(0, N)])
            rowbuf[pl.ds(N, 16)] = jnp.full((16,), 1000000.0, jnp.float32)

            def q_loop(q, _):
                pltpu.sync_copy(rep_hbm.at[b, r, pl.ds(q * GQ, GQ)], repbuf)

                def j_loop(j0, _):
                    for u in range(8):
                        j = j0 * 8 + u
                        iv = idxv[pl.ds(q * GQ + j * 16, 16)]
                        g = plsc.load_gather(rowbuf, [iv])
                        rv = repbuf[pl.ds(j * 16, 16)]
                        gxv = g - rv
                        gxbuf[pl.ds(j * 16, 16)] = gxv
                        nfv = jnp.where(gxv > 100000.0, 0.0, gxv) * 4.0
                        outbuf[pl.ds(j * 16, 16)] = nfv
                    return 0

                lax.fori_loop(0, GQ // 128, j_loop, 0)
                pltpu.sync_copy(outbuf, nf_hbm.at[b, r, pl.ds(q * GQ, GQ)])
                pltpu.sync_copy(gxbuf, gx_hbm.at[b, r, pl.ds(q * GQ, GQ)])
                return 0

            lax.fori_loop(0, PS // GQ, q_loop, 0)

        @pl.when(r >= 3)
        def _feat_row():
            pltpu.sync_copy(feat_hbm.at[b, r - 3], rowbuf.at[pl.ds(0, N)])
            rowbuf[pl.ds(N, 16)] = jnp.zeros((16,), jnp.float32)

            def q_loop(q, _):
                def j_loop(j0, _):
                    for u in range(8):
                        j = j0 * 8 + u
                        iv = idxv[pl.ds(q * GQ + j * 16, 16)]
                        g = plsc.load_gather(rowbuf, [iv])
                        outbuf[pl.ds(j * 16, 16)] = g
                    return 0

                lax.fori_loop(0, GQ // 128, j_loop, 0)
                pltpu.sync_copy(outbuf, nf_hbm.at[b, r, pl.ds(q * GQ, GQ)])
                return 0

            lax.fori_loop(0, PS // GQ, q_loop, 0)

        return 0

    lax.fori_loop(0, nrows, row_loop, 0)


def _gather(features, xyz_t, rep, idxm):
    mesh = plsc.VectorSubcoreMesh(core_axis_name="c", subcore_axis_name="s", num_cores=NC, num_subcores=NS)
    return pl.kernel(
        _gather_body,
        compiler_params=pltpu.CompilerParams(needs_layout_passes=False),
        out_type=(
            jax.ShapeDtypeStruct((B, 3 + C, PS), jnp.float32),
            jax.ShapeDtypeStruct((B, 3, PS), jnp.float32),
        ),
        mesh=mesh,
        scratch_types=[
            pltpu.VMEM((PS,), jnp.int32),
            pltpu.VMEM((N + 16,), jnp.float32),
            pltpu.VMEM((GQ,), jnp.float32),
            pltpu.VMEM((GQ,), jnp.float32),
            pltpu.VMEM((GQ,), jnp.float32),
        ],
    )(features, xyz_t, rep, idxm)


@jax.jit
def kernel(xyz, new_xyz, features):
    xyz_t = jnp.transpose(xyz, (0, 2, 1))
    mask = _compute_mask(new_xyz, xyz_t)
    bq, idxm = _select(mask)
    rep = jnp.broadcast_to(
        jnp.transpose(new_xyz, (0, 2, 1))[..., None],
        (B, 3, P, NSAMPLE)).reshape(B, 3, PS)
    nf3, gx3 = _gather(features, xyz_t, rep, idxm)
    new_features = nf3.reshape(B, 3 + C, P, NSAMPLE)
    grouped_xyz = gx3.reshape(B, 3, P, NSAMPLE)
    bq_idx = bq.astype(jnp.int64)
    return (new_features, grouped_xyz, bq_idx)


# speedup vs baseline: 20.4227x; 1.3936x over previous
import functools

import jax
import jax.numpy as jnp
from jax import lax
from jax.experimental import pallas as pl
from jax.experimental.pallas import tpu as pltpu
from jax.experimental.pallas import tpu_sc as plsc

RADIUS = 0.25
NSAMPLE = 32
B, N, P, C = 4, 8192, 1024, 128
PS = P * NSAMPLE

NC, NS = 2, 16
NW = NC * NS
ROWS_PER_W = (B * P) // NW
SEL_CHUNK = 8
GQ = 8192


def _mask_body(new_ref, xyzt_ref, mask_ref):
    nxyz = new_ref[0]
    x = xyzt_ref[0]
    n2 = (nxyz[:, 0:1] * nxyz[:, 0:1] + nxyz[:, 1:2] * nxyz[:, 1:2]
          + nxyz[:, 2:3] * nxyz[:, 2:3])
    x2 = (x[0:1] * x[0:1] + x[1:2] * x[1:2] + x[2:3] * x[2:3])
    inner = jnp.dot(nxyz, x, preferred_element_type=jnp.float32)
    dist2 = n2 + x2 - 2.0 * inner
    within = dist2 <= (RADIUS * RADIUS)
    mask_ref[...] = within.astype(jnp.int32)


def _compute_mask(new_xyz, xyz_t):
    NB = 2048
    grid = (B, N // NB)
    return pl.pallas_call(
        _mask_body,
        grid=grid,
        in_specs=[
            pl.BlockSpec((1, P, 3), lambda b, n: (b, 0, 0)),
            pl.BlockSpec((1, 3, NB), lambda b, n: (b, 0, n)),
        ],
        out_specs=pl.BlockSpec((P, NB), lambda b, n: (b, n)),
        out_shape=jax.ShapeDtypeStruct((B * P, N), jnp.int32),
    )(new_xyz, xyz_t)


def _select_body(mask_hbm, bq_hbm, im_hbm, mbuf, bqb, imb, idxbuf):
    cid = lax.axis_index("c")
    sid = lax.axis_index("s")
    wid = sid * NC + cid
    row0 = wid * ROWS_PER_W
    ii = lax.iota(jnp.int32, 16)

    def chunk_loop(ci, _):
        r0 = row0 + ci * SEL_CHUNK
        pltpu.sync_copy(mask_hbm.at[pl.ds(r0, SEL_CHUNK), :], mbuf)

        def row_loop(rj, _):
            def do_block(k0, cnt):
                vs = [mbuf[rj, pl.ds((k0 * 8 + u) * 16, 16)]
                      for u in range(8)]
                t01 = vs[0] | vs[1]
                t23 = vs[2] | vs[3]
                t45 = vs[4] | vs[5]
                t67 = vs[6] | vs[7]
                t = (t01 | t23) | (t45 | t67)

                def extract():
                    c = cnt
                    for u in range(8):
                        m = vs[u] > 0
                        pos = (k0 * 8 + u) * 16 + ii
                        pc = plsc.all_reduce_population_count(m)[0]

                        def one_vreg(c=c, m=m, pos=pos, pc=pc):
                            off = jnp.minimum(c, 144)
                            plsc.store_compressed(
                                idxbuf.at[pl.ds(off, 16)], pos, mask=m)
                            return c + pc

                        c = lax.cond(pc > 0, one_vreg, lambda c=c: c)
                    return c

                tpc = plsc.all_reduce_population_count(t > 0)[0]
                return lax.cond(tpc > 0, extract, lambda: cnt)

            def blk_loop(k0, cnt):
                return lax.cond(cnt < NSAMPLE,
                                lambda: do_block(k0, cnt),
                                lambda: cnt)

            cnt = lax.fori_loop(0, 64, blk_loop, jnp.int32(0))
            cnt_c = jnp.minimum(cnt, NSAMPLE)
            head = idxbuf[pl.ds(0, 16)]
            first = jnp.where(cnt > 0, head[0], 0)
            eff = jnp.maximum(cnt_c, 1)
            for s2 in range(2):
                sl = ii + (16 * s2)
                cur = idxbuf[pl.ds(16 * s2, 16)]
                fsp = jnp.broadcast_to(first, (16,))
                bqv = jnp.where(sl < cnt_c, cur, fsp)
                imv = jnp.where(sl < eff, bqv, N)
                rloc = ci * SEL_CHUNK + rj
                bqb[rloc, pl.ds(16 * s2, 16)] = bqv
                imb[pl.ds(rloc * NSAMPLE + 16 * s2, 16)] = imv
            return 0

        lax.fori_loop(0, SEL_CHUNK, row_loop, 0)
        return 0

    lax.fori_loop(0, ROWS_PER_W // SEL_CHUNK, chunk_loop, 0)
    b_w = row0 // P
    p0 = row0 % P
    pltpu.sync_copy(bqb, bq_hbm.at[b_w, pl.ds(p0, ROWS_PER_W), :])
    pltpu.sync_copy(imb, im_hbm.at[b_w, pl.ds(p0 * NSAMPLE,
                                              ROWS_PER_W * NSAMPLE)])


def _select(mask2d):
    mesh = plsc.VectorSubcoreMesh(core_axis_name="c", subcore_axis_name="s", num_cores=NC, num_subcores=NS)
    return pl.kernel(
        _select_body,
        compiler_params=pltpu.CompilerParams(needs_layout_passes=False),
        out_type=(
            jax.ShapeDtypeStruct((B, P, NSAMPLE), jnp.int32),
            jax.ShapeDtypeStruct((B, PS), jnp.int32),
        ),
        mesh=mesh,
        scratch_types=[
            pltpu.VMEM((SEL_CHUNK, N), jnp.int32),
            pltpu.VMEM((ROWS_PER_W, NSAMPLE), jnp.int32),
            pltpu.VMEM((ROWS_PER_W * NSAMPLE,), jnp.int32),
            pltpu.VMEM((176,), jnp.int32),
        ],
    )(mask2d)


def _gather_body(feat_hbm, xyzt_hbm, rep_hbm, idx_hbm,
                 nf_hbm, gx_hbm,
                 idxv, rowbuf, repbuf, outbuf, gxbuf, fullout):
    cid = lax.axis_index("c")
    sid = lax.axis_index("s")
    wid = sid * NC + cid
    b = wid // 8
    s8 = wid % 8
    nrows = jnp.where(s8 < 3, 17, 16)
    rstart = 16 * s8 + jnp.minimum(s8, 3)

    pltpu.sync_copy(idx_hbm.at[b], idxv)

    def row_loop(ri, _):
        r = rstart + ri

        @pl.when(r < 3)
        def _xyz_row():
            pltpu.sync_copy(xyzt_hbm.at[b, r], rowbuf.at[pl.ds(0, N)])
            rowbuf[pl.ds(N, 16)] = jnp.full((16,), 1000000.0, jnp.float32)

            def q_loop(q, _):
                pltpu.sync_copy(rep_hbm.at[b, r, pl.ds(q * GQ, GQ)], repbuf)

                def j_loop(j0, _):
                    for u in range(8):
                        j = j0 * 8 + u
                        iv = idxv[pl.ds(q * GQ + j * 16, 16)]
                        g = plsc.load_gather(rowbuf, [iv])
                        rv = repbuf[pl.ds(j * 16, 16)]
                        gxv = g - rv
                        gxbuf[pl.ds(j * 16, 16)] = gxv
                        nfv = jnp.where(gxv > 100000.0, 0.0, gxv) * 4.0
                        outbuf[pl.ds(j * 16, 16)] = nfv
                    return 0

                lax.fori_loop(0, GQ // 128, j_loop, 0)
                pltpu.sync_copy(outbuf, nf_hbm.at[b, r, pl.ds(q * GQ, GQ)])
                pltpu.sync_copy(gxbuf, gx_hbm.at[b, r, pl.ds(q * GQ, GQ)])
                return 0

            lax.fori_loop(0, PS // GQ, q_loop, 0)

        @pl.when(r >= 3)
        def _feat_row():
            pltpu.sync_copy(feat_hbm.at[b, r - 3], rowbuf.at[pl.ds(0, N)])
            rowbuf[pl.ds(N, 16)] = jnp.zeros((16,), jnp.float32)

            def j_loop(j0, _):
                for u in range(8):
                    j = j0 * 8 + u
                    iv = idxv[pl.ds(j * 16, 16)]
                    g = plsc.load_gather(rowbuf, [iv])
                    fullout[pl.ds(j * 16, 16)] = g
                return 0

            lax.fori_loop(0, PS // 128, j_loop, 0)
            pltpu.sync_copy(fullout, nf_hbm.at[b, r])

        return 0

    lax.fori_loop(0, nrows, row_loop, 0)


def _gather(features, xyz_t, rep, idxm):
    mesh = plsc.VectorSubcoreMesh(core_axis_name="c", subcore_axis_name="s", num_cores=NC, num_subcores=NS)
    return pl.kernel(
        _gather_body,
        compiler_params=pltpu.CompilerParams(needs_layout_passes=False),
        out_type=(
            jax.ShapeDtypeStruct((B, 3 + C, PS), jnp.float32),
            jax.ShapeDtypeStruct((B, 3, PS), jnp.float32),
        ),
        mesh=mesh,
        scratch_types=[
            pltpu.VMEM((PS,), jnp.int32),
            pltpu.VMEM((N + 16,), jnp.float32),
            pltpu.VMEM((GQ,), jnp.float32),
            pltpu.VMEM((GQ,), jnp.float32),
            pltpu.VMEM((GQ,), jnp.float32),
            pltpu.VMEM((PS,), jnp.float32),
        ],
    )(features, xyz_t, rep, idxm)


@jax.jit
def kernel(xyz, new_xyz, features):
    xyz_t = jnp.transpose(xyz, (0, 2, 1))
    mask = _compute_mask(new_xyz, xyz_t)
    bq, idxm = _select(mask)
    rep = jnp.broadcast_to(
        jnp.transpose(new_xyz, (0, 2, 1))[..., None],
        (B, 3, P, NSAMPLE)).reshape(B, 3, PS)
    nf3, gx3 = _gather(features, xyz_t, rep, idxm)
    new_features = nf3.reshape(B, 3 + C, P, NSAMPLE)
    grouped_xyz = gx3.reshape(B, 3, P, NSAMPLE)
    bq_idx = bq.astype(jnp.int64)
    return (new_features, grouped_xyz, bq_idx)
